# Initial kernel scaffold; baseline (speedup 1.0000x reference)
import jax
import jax.numpy as jnp
from jax.experimental import pallas as pl


def kernel(nodes, num_nodes, edges, num_edges, edges_features, emb, W_me1, b_me1, W_me2, b_me2, W_mn1, b_mn1, W_mn2, b_mn2, W_st1, b_st1, W_st2, b_st2, W_ro1, b_ro1, W_ro2, b_ro2):
    raise NotImplementedError("write your pallas kernel here")


# speedup vs baseline: 2.2413x; 2.2413x over previous
import functools

import jax
import jax.numpy as jnp
from jax import lax
from jax.experimental import pallas as pl
from jax.experimental.pallas import tpu as pltpu
from jax.experimental.pallas import tpu_sc as plsc

B = 4
NG = 2500
EG = 80000
N = B * NG
E = B * EG
H = 128
L = 3
ES = 50
ESP = 64
CUTOFF = 5.0
STEP = 0.1
NUM_EMB = 119
LOG2 = 0.6931471805599453
SCALE_EPS = 1e-6

NC = 2
NS = 16
NW = NC * NS
EPW = E // NW
CH = 80
NCHUNK = EPW // CH
RPS = N // NS


def _ssp(x):
    return jnp.maximum(x, 0.0) + jnp.log(1.0 + jnp.exp(-jnp.abs(x))) - LOG2


def _sp(x):
    return jnp.maximum(x, 0.0) + jnp.log(1.0 + jnp.exp(-jnp.abs(x)))


def _embed_body(nodes_ref, emb_ref, out_ref):
    ids = nodes_ref[...]
    lane = lax.broadcasted_iota(jnp.int32, (1000, H), 1)
    oh = (lane == ids).astype(jnp.float32)
    out_ref[...] = jnp.dot(oh, emb_ref[...],
                           preferred_element_type=jnp.float32)


def _tc_embed(nodes2d, emb_pad):
    return pl.pallas_call(
        _embed_body,
        grid=(N // 1000,),
        in_specs=[
            pl.BlockSpec((1000, 1), lambda i: (i, 0)),
            pl.BlockSpec((H, H), lambda i: (0, 0)),
        ],
        out_specs=pl.BlockSpec((1000, H), lambda i: (i, 0)),
        out_shape=jax.ShapeDtypeStruct((N, H), jnp.float32),
    )(nodes2d, emb_pad)


def _gates_body(ef_ref, w1_ref, b1_ref, w2_ref, b2_ref, out_ref):
    x = ef_ref[...]
    k = lax.broadcasted_iota(jnp.int32, (512, ESP), 1).astype(jnp.float32) * STEP
    es = jnp.exp(-((x - k) ** 2) * (1.0 / (2.0 * STEP * STEP)))
    h1 = lax.dot_general(es, w1_ref[...], (((1,), (1,)), ((), ())),
                         preferred_element_type=jnp.float32) + b1_ref[...]
    g = lax.dot_general(_ssp(h1), w2_ref[...], (((1,), (1,)), ((), ())),
                        preferred_element_type=jnp.float32) + b2_ref[...]
    out_ref[...] = g


def _tc_gates(ef2d, w1p, b1, w2, b2):
    return pl.pallas_call(
        _gates_body,
        grid=(E // 512,),
        in_specs=[
            pl.BlockSpec((512, 1), lambda i: (i, 0)),
            pl.BlockSpec((H, ESP), lambda i: (0, 0)),
            pl.BlockSpec((1, H), lambda i: (0, 0)),
            pl.BlockSpec((H, H), lambda i: (0, 0)),
            pl.BlockSpec((1, H), lambda i: (0, 0)),
        ],
        out_specs=pl.BlockSpec((512, H), lambda i: (i, 0)),
        out_shape=jax.ShapeDtypeStruct((E, H), jnp.float32),
    )(ef2d, w1p, b1, w2, b2)


def _mlp_body(x_ref, w1_ref, b1_ref, w2_ref, b2_ref, out_ref):
    x = x_ref[...]
    h1 = lax.dot_general(x, w1_ref[...], (((1,), (1,)), ((), ())),
                         preferred_element_type=jnp.float32) + b1_ref[...]
    out_ref[...] = lax.dot_general(_ssp(h1), w2_ref[...],
                                   (((1,), (1,)), ((), ())),
                                   preferred_element_type=jnp.float32) + b2_ref[...]


def _tc_mlp(x, w1, b1, w2, b2):
    return pl.pallas_call(
        _mlp_body,
        grid=(N // 1000,),
        in_specs=[
            pl.BlockSpec((1000, H), lambda i: (i, 0)),
            pl.BlockSpec((H, H), lambda i: (0, 0)),
            pl.BlockSpec((1, H), lambda i: (0, 0)),
            pl.BlockSpec((H, H), lambda i: (0, 0)),
            pl.BlockSpec((1, H), lambda i: (0, 0)),
        ],
        out_specs=pl.BlockSpec((1000, H), lambda i: (i, 0)),
        out_shape=jax.ShapeDtypeStruct((N, H), jnp.float32),
    )(x, w1, b1, w2, b2)


def _update_body(p_ref, h_ref, w1_ref, b1_ref, w2_ref, b2_ref, out_ref):
    m = p_ref[0] + p_ref[1]
    h1 = lax.dot_general(m, w1_ref[...], (((1,), (1,)), ((), ())),
                         preferred_element_type=jnp.float32) + b1_ref[...]
    out_ref[...] = h_ref[...] + lax.dot_general(
        _ssp(h1), w2_ref[...], (((1,), (1,)), ((), ())),
        preferred_element_type=jnp.float32) + b2_ref[...]


def _tc_update(parts, h, w1, b1, w2, b2):
    return pl.pallas_call(
        _update_body,
        grid=(N // 1000,),
        in_specs=[
            pl.BlockSpec((2, 1000, H), lambda i: (0, i, 0)),
            pl.BlockSpec((1000, H), lambda i: (i, 0)),
            pl.BlockSpec((H, H), lambda i: (0, 0)),
            pl.BlockSpec((1, H), lambda i: (0, 0)),
            pl.BlockSpec((H, H), lambda i: (0, 0)),
            pl.BlockSpec((1, H), lambda i: (0, 0)),
        ],
        out_specs=pl.BlockSpec((1000, H), lambda i: (i, 0)),
        out_shape=jax.ShapeDtypeStruct((N, H), jnp.float32),
    )(parts, h, w1, b1, w2, b2)


def _readout_body(h_ref, w1_ref, b1_ref, w2_ref, b2_ref, out_ref):
    h = h_ref[...]
    h1 = lax.dot_general(h, w1_ref[...], (((1,), (1,)), ((), ())),
                         preferred_element_type=jnp.float32) + b1_ref[...]
    y = _ssp(h1)
    g = lax.broadcasted_iota(jnp.int32, (8, N), 0)
    n = lax.broadcasted_iota(jnp.int32, (8, N), 1) // NG
    sel = (g == n).astype(jnp.float32)
    gsum = jnp.dot(sel, y, preferred_element_type=jnp.float32)
    go = lax.dot_general(gsum, w2_ref[...], (((1,), (1,)), ((), ())),
                         preferred_element_type=jnp.float32) + float(NG) * b2_ref[...]
    col = lax.broadcasted_iota(jnp.int32, (8, H), 1)
    out_ref[...] = jnp.where(col == 1, _sp(go) + SCALE_EPS, go)


def _tc_readout(h, w1, b1, w2p, b2p):
    return pl.pallas_call(
        _readout_body,
        grid=(1,),
        in_specs=[
            pl.BlockSpec((N, H), lambda i: (0, 0)),
            pl.BlockSpec((H, H), lambda i: (0, 0)),
            pl.BlockSpec((1, H), lambda i: (0, 0)),
            pl.BlockSpec((H, H), lambda i: (0, 0)),
            pl.BlockSpec((1, H), lambda i: (0, 0)),
        ],
        out_specs=pl.BlockSpec((8, H), lambda i: (0, 0)),
        out_shape=jax.ShapeDtypeStruct((8, H), jnp.float32),
    )(h, w1, b1, w2p, b2p)


_sc_mesh = plsc.VectorSubcoreMesh(core_axis_name="c", subcore_axis_name="s")


@functools.partial(
    pl.kernel,
    out_type=jax.ShapeDtypeStruct((NC, N, H), jnp.float32),
    mesh=_sc_mesh,
    scratch_types=[
        pltpu.VMEM((CH,), jnp.int32),
        pltpu.VMEM((CH,), jnp.int32),
        pltpu.VMEM((CH, H), jnp.float32),
        pltpu.VMEM((CH, H), jnp.float32),
        pltpu.VMEM((128, H), jnp.float32),
        pltpu.VMEM_SHARED((N, H), jnp.float32),
        pltpu.SemaphoreType.DMA,
    ],
)
def _sc_scatter(a_hbm, g_hbm, src_hbm, dst_hbm, out_hbm,
                srcv, dstv, av, gv, zv, acc, sem):
    c = lax.axis_index("c")
    s = lax.axis_index("s")
    wid = s * NC + c

    zero16 = jnp.zeros((16,), jnp.float32)

    def _zrow(r, _):
        for k in range(H // 16):
            zv[r, pl.ds(k * 16, 16)] = zero16
        return 0

    lax.fori_loop(0, 128, _zrow, 0)
    base0 = s * 624
    for t in range(4):
        pltpu.sync_copy(zv, acc.at[pl.ds(base0 + t * 128, 128)])
    pltpu.sync_copy(zv.at[pl.ds(0, 112)], acc.at[pl.ds(base0 + 512, 112)])

    @pl.when(s == NS - 1)
    def _ztail():
        pltpu.sync_copy(zv.at[pl.ds(0, 16)], acc.at[pl.ds(9984, 16)])

    plsc.subcore_barrier()

    def _chunk(j, _):
        base = wid * EPW + j * CH
        pltpu.sync_copy(src_hbm.at[pl.ds(base, CH)], srcv)
        pltpu.sync_copy(dst_hbm.at[pl.ds(base, CH)], dstv)
        pltpu.sync_copy(g_hbm.at[pl.ds(base, CH)], gv)
        pltpu.async_copy(a_hbm.at[srcv], av, sem).wait()

        def _mrow(r, _):
            for k in range(H // 16):
                sl = pl.ds(k * 16, 16)
                av[r, sl] = av[r, sl] * gv[r, sl]
            return 0

        lax.fori_loop(0, CH, _mrow, 0)
        pltpu.sync_copy(av, acc.at[dstv], add=True)
        return 0

    lax.fori_loop(0, NCHUNK, _chunk, 0)
    plsc.subcore_barrier()

    @pl.when(s == 0)
    def _writeback():
        pltpu.sync_copy(acc, out_hbm.at[c])


def kernel(nodes, num_nodes, edges, num_edges, edges_features, emb,
           W_me1, b_me1, W_me2, b_me2, W_mn1, b_mn1, W_mn2, b_mn2,
           W_st1, b_st1, W_st2, b_st2, W_ro1, b_ro1, W_ro2, b_ro2):
    nodes2d = nodes.reshape(N, 1)
    ef2d = edges_features.reshape(E, 1)
    off = (jnp.arange(B, dtype=jnp.int32) * NG)[:, None, None]
    ecat = (edges + off).reshape(E, 2)
    src = ecat[:, 0]
    dst = ecat[:, 1]

    emb_pad = jnp.pad(emb, ((0, H - NUM_EMB), (0, 0)))
    w_me1_pad = jnp.pad(W_me1, ((0, 0), (0, 0), (0, ESP - ES)))
    w_ro2_pad = jnp.pad(W_ro2, ((0, H - 2), (0, 0)))
    b_ro2_pad = jnp.pad(b_ro2, ((0, H - 2),)).reshape(1, H)

    h = _tc_embed(nodes2d, emb_pad)
    gs = [
        _tc_gates(ef2d, w_me1_pad[i], b_me1[i].reshape(1, H),
                  W_me2[i], b_me2[i].reshape(1, H))
        for i in range(L)
    ]
    for i in range(L):
        a = _tc_mlp(h, W_mn1[i], b_mn1[i].reshape(1, H),
                    W_mn2[i], b_mn2[i].reshape(1, H))
        parts = _sc_scatter(a, gs[i], src, dst)
        h = _tc_update(parts, h, W_st1[i], b_st1[i].reshape(1, H),
                       W_st2[i], b_st2[i].reshape(1, H))

    go = _tc_readout(h, W_ro1, b_ro1.reshape(1, H), w_ro2_pad, b_ro2_pad)
    loc = go[0:B, 0:1]
    scale = go[0:B, 1:2]
    return (loc, scale)
